# baseline (speedup 1.0000x reference)
import functools

import jax
import jax.numpy as jnp
from jax.experimental import pallas as pl
from jax.experimental.pallas import tpu as pltpu

NUM_HEADS = 16
HEAD_DIM = 128


def _matmul_kernel(x_ref, w_ref, o_ref):
    acc = jax.lax.dot_general(
        x_ref[...], w_ref[...],
        dimension_numbers=(((1,), (1,)), ((), ())),
        preferred_element_type=jnp.float32)
    o_ref[...] = acc.astype(o_ref.dtype)


def _matmul_bias_kernel(x_ref, w_ref, b_ref, o_ref):
    acc = jax.lax.dot_general(
        x_ref[...], w_ref[...],
        dimension_numbers=(((1,), (1,)), ((), ())),
        preferred_element_type=jnp.float32) + b_ref[...]
    o_ref[...] = acc.astype(o_ref.dtype)


def _matmul_t(x, w, bm, bn, bias=None, out_dtype=jnp.float32,
              interpret=False):
    m, k = x.shape
    n, k2 = w.shape
    assert k == k2 and m % bm == 0 and n % bn == 0
    grid = (m // bm, n // bn)
    x_spec = pl.BlockSpec((bm, k), lambda i, j: (i, 0))
    w_spec = pl.BlockSpec((bn, k), lambda i, j: (j, 0))
    o_spec = pl.BlockSpec((bm, bn), lambda i, j: (i, j))
    out_type = jax.ShapeDtypeStruct((m, n), out_dtype)
    if bias is None:
        return pl.pallas_call(
            _matmul_kernel, grid=grid,
            in_specs=[x_spec, w_spec], out_specs=o_spec,
            out_shape=out_type, interpret=interpret)(x, w)
    b_spec = pl.BlockSpec((1, bn), lambda i, j: (0, j))
    return pl.pallas_call(
        _matmul_bias_kernel, grid=grid,
        in_specs=[x_spec, w_spec, b_spec], out_specs=o_spec,
        out_shape=out_type, interpret=interpret)(x, w, bias.reshape(1, n))


def _flash_kernel(q_ref, k_ref, v_ref, o_ref, s_scr, *, bq, bk, scale):
    qi = pl.program_id(1)
    q = q_ref[0]

    def logits(j):
        kb = k_ref[0, pl.ds(j * bk, bk), :]
        s = jax.lax.dot_general(
            q, kb, dimension_numbers=(((1,), (1,)), ((), ())),
            preferred_element_type=jnp.float32)
        return s * scale

    def pass_a(j, m):
        s = logits(j)
        s_scr[:, pl.ds(j * bk, bk)] = s
        return jnp.maximum(m, jnp.max(s, axis=1, keepdims=True))

    m = jax.lax.fori_loop(
        0, qi, pass_a, jnp.full((bq, 1), -jnp.inf, jnp.float32))
    s = logits(qi)
    rows = jax.lax.broadcasted_iota(jnp.int32, (bq, bk), 0)
    cols = jax.lax.broadcasted_iota(jnp.int32, (bq, bk), 1)
    s = jnp.where(cols <= rows, s, -jnp.inf)
    s_scr[:, pl.ds(qi * bk, bk)] = s
    m = jnp.maximum(m, jnp.max(s, axis=1, keepdims=True))

    def pass_b(j, carry):
        l, acc = carry
        p = jnp.exp(s_scr[:, pl.ds(j * bk, bk)] - m)
        l = l + jnp.sum(p, axis=1, keepdims=True)
        vb = v_ref[0, pl.ds(j * bk, bk), :]
        acc = acc + jnp.dot(p.astype(jnp.bfloat16), vb,
                            preferred_element_type=jnp.float32)
        return l, acc

    l, acc = jax.lax.fori_loop(
        0, qi + 1, pass_b, (jnp.zeros((bq, 1), jnp.float32),
                            jnp.zeros((bq, HEAD_DIM), jnp.float32)))
    o_ref[0] = (acc / l).astype(o_ref.dtype)


def _flash_attention(qkv, bq, bk, interpret=False):
    b, s, three_hidden = qkv.shape
    hidden = three_hidden // 3
    scale = 1.0 / (HEAD_DIM ** 0.5)
    grid = (b * NUM_HEADS, s // bq)
    q_spec = pl.BlockSpec(
        (1, bq, HEAD_DIM),
        lambda bh, qi: (bh // NUM_HEADS, qi, bh % NUM_HEADS))
    k_spec = pl.BlockSpec(
        (1, s, HEAD_DIM),
        lambda bh, qi: (bh // NUM_HEADS, 0, NUM_HEADS + bh % NUM_HEADS))
    v_spec = pl.BlockSpec(
        (1, s, HEAD_DIM),
        lambda bh, qi: (bh // NUM_HEADS, 0, 2 * NUM_HEADS + bh % NUM_HEADS))
    o_spec = pl.BlockSpec(
        (1, bq, HEAD_DIM),
        lambda bh, qi: (bh // NUM_HEADS, qi, bh % NUM_HEADS))
    return pl.pallas_call(
        functools.partial(_flash_kernel, bq=bq, bk=bk, scale=scale),
        grid=grid,
        in_specs=[q_spec, k_spec, v_spec],
        out_specs=o_spec,
        out_shape=jax.ShapeDtypeStruct((b, s, hidden), jnp.bfloat16),
        scratch_shapes=[pltpu.VMEM((bq, s), jnp.float32)],
        interpret=interpret)(qkv, qkv, qkv)


def kernel(x, Wq, Wk, Wv, Wo, bo, interpret=False):
    b, s, hidden = x.shape
    wc = jnp.concatenate([Wq, Wk, Wv], axis=0).astype(jnp.bfloat16)
    x2 = x.reshape(b * s, hidden).astype(jnp.bfloat16)
    qkv = _matmul_t(x2, wc, bm=1024, bn=512, out_dtype=jnp.bfloat16,
                    interpret=interpret)
    qkv = qkv.reshape(b, s, 3 * hidden)
    attn = _flash_attention(qkv, bq=512, bk=512, interpret=interpret)
    out = _matmul_t(attn.reshape(b * s, hidden), Wo.astype(jnp.bfloat16),
                    bm=1024, bn=512, bias=bo, out_dtype=jnp.float32,
                    interpret=interpret)
    return out.reshape(b, s, hidden)


# speedup vs baseline: 1.1017x; 1.1017x over previous
import functools

import jax
import jax.numpy as jnp
from jax.experimental import pallas as pl
from jax.experimental.pallas import tpu as pltpu

NUM_HEADS = 16
HEAD_DIM = 128


def _cast_kernel(a_ref, b_ref, c_ref, d_ref, oa_ref, ob_ref, oc_ref, od_ref):
    oa_ref[...] = a_ref[...].astype(jnp.bfloat16)
    ob_ref[...] = b_ref[...].astype(jnp.bfloat16)
    oc_ref[...] = c_ref[...].astype(jnp.bfloat16)
    od_ref[...] = d_ref[...].astype(jnp.bfloat16)


def _cast_weights(wq, wk, wv, wo, interpret=False):
    n, k = wq.shape
    bm = 256
    spec = pl.BlockSpec((bm, k), lambda i: (i, 0))
    out = jax.ShapeDtypeStruct((n, k), jnp.bfloat16)
    return pl.pallas_call(
        _cast_kernel, grid=(n // bm,),
        in_specs=[spec] * 4, out_specs=[spec] * 4,
        out_shape=[out] * 4, interpret=interpret)(wq, wk, wv, wo)


def _qkv_kernel(x_ref, wq_ref, wk_ref, wv_ref, q_ref, k_ref, v_ref):
    xb = x_ref[...].astype(jnp.bfloat16)
    dn = (((1,), (1,)), ((), ()))
    q_ref[...] = jax.lax.dot_general(
        xb, wq_ref[...], dn, preferred_element_type=jnp.float32
    ).astype(jnp.bfloat16)
    k_ref[...] = jax.lax.dot_general(
        xb, wk_ref[...], dn, preferred_element_type=jnp.float32
    ).astype(jnp.bfloat16)
    v_ref[...] = jax.lax.dot_general(
        xb, wv_ref[...], dn, preferred_element_type=jnp.float32
    ).astype(jnp.bfloat16)


def _qkv_proj(x2, wqb, wkb, wvb, bm, interpret=False):
    m, k = x2.shape
    n = wqb.shape[0]
    x_spec = pl.BlockSpec((bm, k), lambda i: (i, 0))
    w_spec = pl.BlockSpec((n, k), lambda i: (0, 0))
    o_spec = pl.BlockSpec((bm, n), lambda i: (i, 0))
    out = jax.ShapeDtypeStruct((m, n), jnp.bfloat16)
    return pl.pallas_call(
        _qkv_kernel, grid=(m // bm,),
        in_specs=[x_spec, w_spec, w_spec, w_spec],
        out_specs=[o_spec] * 3,
        out_shape=[out] * 3, interpret=interpret)(x2, wqb, wkb, wvb)


def _out_kernel(a_ref, w_ref, b_ref, o_ref):
    acc = jax.lax.dot_general(
        a_ref[...], w_ref[...], (((1,), (1,)), ((), ())),
        preferred_element_type=jnp.float32)
    o_ref[...] = acc + b_ref[...]


def _out_proj(attn2, wob, bo, bm, interpret=False):
    m, k = attn2.shape
    n = wob.shape[0]
    a_spec = pl.BlockSpec((bm, k), lambda i: (i, 0))
    w_spec = pl.BlockSpec((n, k), lambda i: (0, 0))
    b_spec = pl.BlockSpec((1, n), lambda i: (0, 0))
    o_spec = pl.BlockSpec((bm, n), lambda i: (i, 0))
    return pl.pallas_call(
        _out_kernel, grid=(m // bm,),
        in_specs=[a_spec, w_spec, b_spec],
        out_specs=o_spec,
        out_shape=jax.ShapeDtypeStruct((m, n), jnp.float32),
        interpret=interpret)(attn2, wob, bo.reshape(1, n))


def _flash_kernel(q_ref, k_ref, v_ref, o_ref, s_scr, *, bq, bk, scale):
    qi = pl.program_id(1)
    q = q_ref[0]

    def logits(j):
        kb = k_ref[0, pl.ds(j * bk, bk), :]
        s = jax.lax.dot_general(
            q, kb, dimension_numbers=(((1,), (1,)), ((), ())),
            preferred_element_type=jnp.float32)
        return s * scale

    def pass_a(j, m):
        s = logits(j)
        s_scr[:, pl.ds(j * bk, bk)] = s
        return jnp.maximum(m, jnp.max(s, axis=1, keepdims=True))

    m = jax.lax.fori_loop(
        0, qi, pass_a, jnp.full((bq, 1), -jnp.inf, jnp.float32))
    s = logits(qi)
    rows = jax.lax.broadcasted_iota(jnp.int32, (bq, bk), 0)
    cols = jax.lax.broadcasted_iota(jnp.int32, (bq, bk), 1)
    s = jnp.where(cols <= rows, s, -jnp.inf)
    s_scr[:, pl.ds(qi * bk, bk)] = s
    m = jnp.maximum(m, jnp.max(s, axis=1, keepdims=True))

    def pass_b(j, carry):
        l, acc = carry
        p = jnp.exp(s_scr[:, pl.ds(j * bk, bk)] - m)
        l = l + jnp.sum(p, axis=1, keepdims=True)
        vb = v_ref[0, pl.ds(j * bk, bk), :]
        acc = acc + jnp.dot(p.astype(jnp.bfloat16), vb,
                            preferred_element_type=jnp.float32)
        return l, acc

    l, acc = jax.lax.fori_loop(
        0, qi + 1, pass_b, (jnp.zeros((bq, 1), jnp.float32),
                            jnp.zeros((bq, HEAD_DIM), jnp.float32)))
    o_ref[0] = (acc / l).astype(o_ref.dtype)


def _flash_attention(q, k, v, bq, bk, interpret=False):
    b, s, hidden = q.shape
    scale = 1.0 / (HEAD_DIM ** 0.5)
    grid = (b * NUM_HEADS, s // bq)
    q_spec = pl.BlockSpec(
        (1, bq, HEAD_DIM),
        lambda bh, qi: (bh // NUM_HEADS, qi, bh % NUM_HEADS))
    kv_spec = pl.BlockSpec(
        (1, s, HEAD_DIM),
        lambda bh, qi: (bh // NUM_HEADS, 0, bh % NUM_HEADS))
    o_spec = pl.BlockSpec(
        (1, bq, HEAD_DIM),
        lambda bh, qi: (bh // NUM_HEADS, qi, bh % NUM_HEADS))
    return pl.pallas_call(
        functools.partial(_flash_kernel, bq=bq, bk=bk, scale=scale),
        grid=grid,
        in_specs=[q_spec, kv_spec, kv_spec],
        out_specs=o_spec,
        out_shape=jax.ShapeDtypeStruct((b, s, hidden), jnp.bfloat16),
        scratch_shapes=[pltpu.VMEM((bq, s), jnp.float32)],
        interpret=interpret)(q, k, v)


def kernel(x, Wq, Wk, Wv, Wo, bo, interpret=False):
    b, s, hidden = x.shape
    wqb, wkb, wvb, wob = _cast_weights(Wq, Wk, Wv, Wo, interpret=interpret)
    x2 = x.reshape(b * s, hidden)
    q2, k2, v2 = _qkv_proj(x2, wqb, wkb, wvb, bm=512, interpret=interpret)
    q3 = q2.reshape(b, s, hidden)
    k3 = k2.reshape(b, s, hidden)
    v3 = v2.reshape(b, s, hidden)
    attn = _flash_attention(q3, k3, v3, bq=512, bk=512, interpret=interpret)
    out = _out_proj(attn.reshape(b * s, hidden), wob, bo, bm=512,
                    interpret=interpret)
    return out.reshape(b, s, hidden)
